# baseline (speedup 1.0000x reference)
import functools

import jax
import jax.numpy as jnp
from jax import lax
from jax.experimental import pallas as pl
from jax.experimental.pallas import tpu as pltpu

T_PER_K = 4
N_NEIGH_TRI_K = 20
K_KNN_K = 15
U_TRI_K = 2000

_SENTINEL = 1.0e6


_NSLOT = 4


def _fill_strip(points_ref, coords_t_ref, qsq_ref, rsq_ref, d3_ref, i, R, npad, NB):
    p = points_ref[...]
    ct = coords_t_ref[...]
    dotp = lax.dot_general(p, ct, (((1,), (0,)), ((), ())),
                           preferred_element_type=jnp.float32)
    d2 = (qsq_ref[...] + rsq_ref[...]) - 2.0 * dotp
    cols = lax.broadcasted_iota(jnp.int32, (R, npad), 1)
    rows_g = i * R + lax.broadcasted_iota(jnp.int32, (R, npad), 0)
    d2 = jnp.where(cols == rows_g, jnp.inf, d2)
    for b in range(NB):
        d3_ref[b] = d2[:, b * 128:(b + 1) * 128]


def _knn_body(k, n_real, points_ref, coords_t_ref, qsq_ref, rsq_ref, out_ref,
              d3_ref):
    R = points_ref.shape[0]
    npad = coords_t_ref.shape[1]
    NB = npad // 128
    i = pl.program_id(0)
    INF = jnp.inf

    _fill_strip(points_ref, coords_t_ref, qsq_ref, rsq_ref, d3_ref, i, R, npad, NB)

    biota = lax.broadcasted_iota(jnp.int32, (NB, R), 0)
    Ms, As = [], []
    for s in range(_NSLOT):
        d = d3_ref[...]
        lanes = lax.broadcasted_iota(jnp.int32, (NB, R, 128), 2)
        m = jnp.min(d, axis=2)
        a = jnp.min(jnp.where(d == m[:, :, None], lanes, 128), axis=2)
        Ms.append(m)
        As.append(biota * 128 + a)
        if s + 1 < _NSLOT:
            d3_ref[...] = jnp.where(lanes == a[:, :, None], INF, d)

    C = jnp.zeros((NB, R), jnp.int32)
    for t in range(k):
        gm = jnp.min(Ms[0], axis=0, keepdims=True)
        b_star = jnp.min(jnp.where(Ms[0] == gm, biota, NB), axis=0,
                         keepdims=True)
        onb = biota == b_star
        idx = jnp.min(jnp.where(onb, As[0], npad), axis=0, keepdims=True)
        out_ref[t:t + 1, :] = idx
        for s in range(_NSLOT - 1):
            Ms[s] = jnp.where(onb, Ms[s + 1], Ms[s])
            As[s] = jnp.where(onb, As[s + 1], As[s])
        Ms[_NSLOT - 1] = jnp.where(onb, INF, Ms[_NSLOT - 1])
        As[_NSLOT - 1] = jnp.where(onb, npad, As[_NSLOT - 1])
        C = C + onb.astype(jnp.int32)

    bad = jnp.max(C) >= _NSLOT

    @pl.when(bad)
    def _slow():
        _fill_strip(points_ref, coords_t_ref, qsq_ref, rsq_ref, d3_ref,
                    i, R, npad, NB)
        for t in range(k):
            dd = d3_ref[...]
            colg = (lax.broadcasted_iota(jnp.int32, (NB, R, 128), 0) * 128
                    + lax.broadcasted_iota(jnp.int32, (NB, R, 128), 2))
            m2 = jnp.min(dd, axis=2)
            m1 = jnp.min(m2, axis=0, keepdims=True)
            sel = jnp.where(dd == m1[:, :, None], colg, npad)
            i2 = jnp.min(sel, axis=2)
            idx = jnp.min(i2, axis=0, keepdims=True)
            out_ref[t:t + 1, :] = idx
            d3_ref[...] = jnp.where(colg == idx[:, :, None], INF, dd)


def _knn_pallas(points, k, row_block):
    n = points.shape[0]
    npad = ((n + 127) // 128) * 128
    if npad % row_block != 0:
        npad = ((npad + row_block - 1) // row_block) * row_block
    pts = jnp.full((npad, 3), _SENTINEL, dtype=jnp.float32).at[:n].set(points)
    coords_t = pts.T
    sq = jnp.sum(pts * pts, axis=1)
    qsq = sq[:, None]
    rsq = sq[None, :]

    grid = npad // row_block
    out = pl.pallas_call(
        functools.partial(_knn_body, k, n),
        grid=(grid,),
        in_specs=[
            pl.BlockSpec((row_block, 3), lambda i: (i, 0)),
            pl.BlockSpec((3, npad), lambda i: (0, 0)),
            pl.BlockSpec((row_block, 1), lambda i: (i, 0)),
            pl.BlockSpec((1, npad), lambda i: (0, 0)),
        ],
        out_specs=pl.BlockSpec((k, row_block), lambda i: (0, i)),
        out_shape=jax.ShapeDtypeStruct((k, npad), jnp.int32),
        scratch_shapes=[pltpu.VMEM((npad // 128, row_block, 128), jnp.float32)],
    )(pts, coords_t, qsq, rsq)
    return out.T[:n]


def _mlp_body(nn, nblk, *refs):
    g_refs = refs[:nn]
    tbl_ref, W1_ref, w2_ref, fs_ref = refs[nn], refs[nn + 1], refs[nn + 2], refs[nn + 3]
    self16 = tbl_ref[...]
    self12 = self16[:, 0:12]
    selfp = self16[:, 12:13]
    W1 = W1_ref[...]
    s = None
    for j in range(nn):
        g = g_refs[j][...]
        r_j = self12 - g[:, 0:12]
        hm = jnp.maximum(lax.dot_general(r_j, W1, (((1,), (0,)), ((), ())),
                                         preferred_element_type=jnp.float32), 0.0)
        wm = hm * g[:, 12:13]
        s = wm if s is None else s + wm
    aggm = s / float(nn)
    dv = lax.dot_general(aggm, w2_ref[...], (((1,), (0,)), ((), ())),
                         preferred_element_type=jnp.float32)
    fs_ref[...] = jax.nn.sigmoid(dv) * selfp


def _mlp_scores_pallas(tbl, g_perm, W1, w2, nn, row_block=1200):
    T = tbl.shape[0]
    nblk = T // row_block
    in_specs = (
        [pl.BlockSpec((row_block, 16), functools.partial(lambda j, i: (j * nblk + i, 0), j))
         for j in range(nn)]
        + [pl.BlockSpec((row_block, 16), lambda i: (i, 0)),
           pl.BlockSpec((12, 128), lambda i: (0, 0)),
           pl.BlockSpec((128, 1), lambda i: (0, 0))]
    )
    fs = pl.pallas_call(
        functools.partial(_mlp_body, nn, nblk),
        grid=(nblk,),
        in_specs=in_specs,
        out_specs=pl.BlockSpec((row_block, 1), lambda i: (i, 0)),
        out_shape=jax.ShapeDtypeStruct((T, 1), jnp.float32),
    )(*([g_perm] * nn + [tbl, W1, w2[:, None]]))
    return fs[:, 0]


def kernel(user_number_triangles, graph_nodes, graph_adjacency_matrix,
           W_gnn, w_out, W_dev, W_mlp1, w_mlp2):
    n = graph_nodes.shape[0]
    src = graph_adjacency_matrix[0]
    dst = graph_adjacency_matrix[1]
    diff = graph_nodes[src] - graph_nodes[dst]
    h = jax.nn.relu(diff @ W_gnn)
    agg = jax.ops.segment_max(h, dst, num_segments=n)
    agg = jnp.where(jnp.isfinite(agg), agg, 0.0)
    inclusion_score = agg @ w_out
    target = min(n, U_TRI_K * 3)
    _, sel_idx = jax.lax.top_k(inclusion_score, target)
    ext = graph_nodes[sel_idx]

    neigh = _knn_pallas(jax.lax.stop_gradient(ext), K_KNN_K, 128)

    dif2 = ext[:, None, :] - ext[neigh]
    h2 = jax.nn.relu(dif2 @ W_dev)
    edge_feat = jnp.max(h2, axis=1)
    f = jnp.mean(edge_feat, axis=1)
    logits = f[:, None] + f[neigh]
    S = jax.nn.softmax(logits, axis=1)
    A_s = S / (jnp.sum(S, axis=1, keepdims=True) + 1e-12)
    idx_i = jnp.repeat(jnp.arange(target), T_PER_K)
    m_t = jnp.tile(jnp.arange(T_PER_K), target)
    tri_j = neigh[idx_i, m_t]
    tri_k = neigh[idx_i, m_t + 1]
    tri_ids = jnp.stack([idx_i, tri_j, tri_k], axis=1)
    triangles = ext[tri_ids]
    p_init = A_s[idx_i, m_t] * A_s[idx_i, m_t + 1]
    bary = jnp.mean(triangles, axis=1)

    n_idx = _knn_pallas(jax.lax.stop_gradient(bary), N_NEIGH_TRI_K, 128)

    T = tri_ids.shape[0]
    tbl = jnp.concatenate(
        [bary, triangles.reshape(T, 9), p_init[:, None],
         jnp.zeros((T, 3), jnp.float32)], axis=1)
    g_perm = tbl[n_idx.T.reshape(-1)]
    final_scores = _mlp_scores_pallas(tbl, g_perm, W_mlp1, w_mlp2, N_NEIGH_TRI_K)
    _, top_tri = jax.lax.top_k(final_scores, U_TRI_K)
    top_tri = top_tri + 0 * user_number_triangles
    return triangles[top_tri]


# speedup vs baseline: 3.5756x; 3.5756x over previous
import functools

import jax
import jax.numpy as jnp
from jax import lax
from jax.experimental import pallas as pl
from jax.experimental.pallas import tpu as pltpu

T_PER_K = 4
N_NEIGH_TRI_K = 20
K_KNN_K = 15
U_TRI_K = 2000

_SENTINEL = 1.0e6


def _knn_body(k, n_real, points_ref, coords_t_ref, qsq_ref, rsq_ref, out_ref,
              d_ref):
    R = points_ref.shape[0]
    npad = coords_t_ref.shape[1]
    i = pl.program_id(0)

    p = points_ref[...]
    ct = coords_t_ref[...]
    dotp = lax.dot_general(p, ct, (((1,), (0,)), ((), ())),
                           preferred_element_type=jnp.float32)
    d = (qsq_ref[...] + rsq_ref[...]) - 2.0 * dotp

    cols = lax.broadcasted_iota(jnp.int32, (R, npad), 1)
    rows_g = i * R + lax.broadcasted_iota(jnp.int32, (R, npad), 0)
    d_ref[...] = jnp.where(cols == rows_g, jnp.inf, d)

    for t in range(k):
        dcur = d_ref[...]
        m = jnp.min(dcur, axis=1, keepdims=True)
        sel = jnp.where(dcur == m, cols, npad)
        idx = jnp.min(sel, axis=1, keepdims=True)
        out_ref[:, t:t + 1] = idx
        d_ref[...] = jnp.where(cols == idx, jnp.inf, dcur)


def _knn_pallas(points, k, row_block):
    n = points.shape[0]
    npad = ((n + 127) // 128) * 128
    if npad % row_block != 0:
        npad = ((npad + row_block - 1) // row_block) * row_block
    pts = jnp.full((npad, 3), _SENTINEL, dtype=jnp.float32).at[:n].set(points)
    coords_t = pts.T
    sq = jnp.sum(pts * pts, axis=1)
    qsq = sq[:, None]
    rsq = sq[None, :]

    grid = npad // row_block
    out = pl.pallas_call(
        functools.partial(_knn_body, k, n),
        grid=(grid,),
        in_specs=[
            pl.BlockSpec((row_block, 3), lambda i: (i, 0)),
            pl.BlockSpec((3, npad), lambda i: (0, 0)),
            pl.BlockSpec((row_block, 1), lambda i: (i, 0)),
            pl.BlockSpec((1, npad), lambda i: (0, 0)),
        ],
        out_specs=pl.BlockSpec((row_block, k), lambda i: (i, 0)),
        out_shape=jax.ShapeDtypeStruct((npad, k), jnp.int32),
        scratch_shapes=[pltpu.VMEM((row_block, npad), jnp.float32)],
    )(pts, coords_t, qsq, rsq)
    return out[:n]


def _mlp_body(nn, nblk, *refs):
    g_refs = refs[:nn]
    tbl_ref, W1_ref, w2_ref, fs_ref = refs[nn], refs[nn + 1], refs[nn + 2], refs[nn + 3]
    self16 = tbl_ref[...]
    self12 = self16[:, 0:12]
    selfp = self16[:, 12:13]
    W1 = W1_ref[...]
    s = None
    for j in range(nn):
        g = g_refs[j][...]
        r_j = self12 - g[:, 0:12]
        hm = jnp.maximum(lax.dot_general(r_j, W1, (((1,), (0,)), ((), ())),
                                         preferred_element_type=jnp.float32), 0.0)
        wm = hm * g[:, 12:13]
        s = wm if s is None else s + wm
    aggm = s / float(nn)
    dv = lax.dot_general(aggm, w2_ref[...], (((1,), (0,)), ((), ())),
                         preferred_element_type=jnp.float32)
    fs_ref[...] = jax.nn.sigmoid(dv) * selfp


def _mlp_scores_pallas(tbl, g_perm, W1, w2, nn, row_block=1200):
    T = tbl.shape[0]
    nblk = T // row_block
    in_specs = (
        [pl.BlockSpec((row_block, 16), functools.partial(lambda j, i: (j * nblk + i, 0), j))
         for j in range(nn)]
        + [pl.BlockSpec((row_block, 16), lambda i: (i, 0)),
           pl.BlockSpec((12, 128), lambda i: (0, 0)),
           pl.BlockSpec((128, 1), lambda i: (0, 0))]
    )
    fs = pl.pallas_call(
        functools.partial(_mlp_body, nn, nblk),
        grid=(nblk,),
        in_specs=in_specs,
        out_specs=pl.BlockSpec((row_block, 1), lambda i: (i, 0)),
        out_shape=jax.ShapeDtypeStruct((T, 1), jnp.float32),
    )(*([g_perm] * nn + [tbl, W1, w2[:, None]]))
    return fs[:, 0]


def kernel(user_number_triangles, graph_nodes, graph_adjacency_matrix,
           W_gnn, w_out, W_dev, W_mlp1, w_mlp2):
    n = graph_nodes.shape[0]
    src = graph_adjacency_matrix[0]
    dst = graph_adjacency_matrix[1]
    diff = graph_nodes[src] - graph_nodes[dst]
    h = jax.nn.relu(diff @ W_gnn)
    agg = jax.ops.segment_max(h, dst, num_segments=n)
    agg = jnp.where(jnp.isfinite(agg), agg, 0.0)
    inclusion_score = agg @ w_out
    target = min(n, U_TRI_K * 3)
    _, sel_idx = jax.lax.top_k(inclusion_score, target)
    ext = graph_nodes[sel_idx]

    neigh = _knn_pallas(jax.lax.stop_gradient(ext), K_KNN_K, 64)

    dif2 = ext[:, None, :] - ext[neigh]
    h2 = jax.nn.relu(dif2 @ W_dev)
    edge_feat = jnp.max(h2, axis=1)
    f = jnp.mean(edge_feat, axis=1)
    logits = f[:, None] + f[neigh]
    S = jax.nn.softmax(logits, axis=1)
    A_s = S / (jnp.sum(S, axis=1, keepdims=True) + 1e-12)
    idx_i = jnp.repeat(jnp.arange(target), T_PER_K)
    m_t = jnp.tile(jnp.arange(T_PER_K), target)
    tri_j = neigh[idx_i, m_t]
    tri_k = neigh[idx_i, m_t + 1]
    tri_ids = jnp.stack([idx_i, tri_j, tri_k], axis=1)
    triangles = ext[tri_ids]
    p_init = A_s[idx_i, m_t] * A_s[idx_i, m_t + 1]
    bary = jnp.mean(triangles, axis=1)

    n_idx = _knn_pallas(jax.lax.stop_gradient(bary), N_NEIGH_TRI_K, 128)

    T = tri_ids.shape[0]
    tbl = jnp.concatenate(
        [bary, triangles.reshape(T, 9), p_init[:, None],
         jnp.zeros((T, 3), jnp.float32)], axis=1)
    g_perm = tbl[n_idx.T.reshape(-1)]
    final_scores = _mlp_scores_pallas(tbl, g_perm, W_mlp1, w_mlp2, N_NEIGH_TRI_K)
    _, top_tri = jax.lax.top_k(final_scores, U_TRI_K)
    top_tri = top_tri + 0 * user_number_triangles
    return triangles[top_tri]


# speedup vs baseline: 3.8072x; 1.0648x over previous
import functools

import jax
import jax.numpy as jnp
from jax import lax
from jax.experimental import pallas as pl
from jax.experimental.pallas import tpu as pltpu

T_PER_K = 4
N_NEIGH_TRI_K = 20
K_KNN_K = 15
U_TRI_K = 2000

_SENTINEL = 1.0e6


def _knn_body(k, n_real, points_ref, coords_t_ref, qsq_ref, rsq_ref, out_ref,
              d_ref):
    R = points_ref.shape[0]
    npad = coords_t_ref.shape[1]
    i = pl.program_id(0)

    p = points_ref[...]
    ct = coords_t_ref[...]
    dotp = lax.dot_general(p, ct, (((1,), (0,)), ((), ())),
                           preferred_element_type=jnp.float32)
    d = (qsq_ref[...] + rsq_ref[...]) - 2.0 * dotp

    cols = lax.broadcasted_iota(jnp.int32, (R, npad), 1)
    rows_g = i * R + lax.broadcasted_iota(jnp.int32, (R, npad), 0)
    d_ref[...] = jnp.where(cols == rows_g, jnp.inf, d)

    for t in range(k):
        dcur = d_ref[...]
        m = jnp.min(dcur, axis=1, keepdims=True)
        sel = jnp.where(dcur == m, cols, npad)
        idx = jnp.min(sel, axis=1, keepdims=True)
        out_ref[:, t:t + 1] = idx
        d_ref[...] = jnp.where(cols == idx, jnp.inf, dcur)


def _knn_pallas(points, k, row_block):
    n = points.shape[0]
    npad = ((n + 127) // 128) * 128
    if npad % row_block != 0:
        npad = ((npad + row_block - 1) // row_block) * row_block
    pts = jnp.full((npad, 3), _SENTINEL, dtype=jnp.float32).at[:n].set(points)
    coords_t = pts.T
    sq = jnp.sum(pts * pts, axis=1)
    qsq = sq[:, None]
    rsq = sq[None, :]

    grid = npad // row_block
    out = pl.pallas_call(
        functools.partial(_knn_body, k, n),
        grid=(grid,),
        in_specs=[
            pl.BlockSpec((row_block, 3), lambda i: (i, 0)),
            pl.BlockSpec((3, npad), lambda i: (0, 0)),
            pl.BlockSpec((row_block, 1), lambda i: (i, 0)),
            pl.BlockSpec((1, npad), lambda i: (0, 0)),
        ],
        out_specs=pl.BlockSpec((row_block, k), lambda i: (i, 0)),
        out_shape=jax.ShapeDtypeStruct((npad, k), jnp.int32),
        scratch_shapes=[pltpu.VMEM((row_block, npad), jnp.float32)],
    )(pts, coords_t, qsq, rsq)
    return out[:n]


def _mlp_body(nn, nblk, *refs):
    g_refs = refs[:nn]
    tbl_ref, W1_ref, w2_ref, fs_ref = refs[nn], refs[nn + 1], refs[nn + 2], refs[nn + 3]
    self16 = tbl_ref[...]
    self12 = self16[:, 0:12]
    selfp = self16[:, 12:13]
    W1 = W1_ref[...]
    s = None
    for j in range(nn):
        g = g_refs[j][...]
        r_j = self12 - g[:, 0:12]
        hm = jnp.maximum(lax.dot_general(r_j, W1, (((1,), (0,)), ((), ())),
                                         preferred_element_type=jnp.float32), 0.0)
        wm = hm * g[:, 12:13]
        s = wm if s is None else s + wm
    aggm = s / float(nn)
    dv = lax.dot_general(aggm, w2_ref[...], (((1,), (0,)), ((), ())),
                         preferred_element_type=jnp.float32)
    fs_ref[...] = jax.nn.sigmoid(dv) * selfp


def _mlp_scores_pallas(tbl, g_perm, W1, w2, nn, row_block=1200):
    T = tbl.shape[0]
    nblk = T // row_block
    in_specs = (
        [pl.BlockSpec((row_block, 16), functools.partial(lambda j, i: (j * nblk + i, 0), j))
         for j in range(nn)]
        + [pl.BlockSpec((row_block, 16), lambda i: (i, 0)),
           pl.BlockSpec((12, 128), lambda i: (0, 0)),
           pl.BlockSpec((128, 1), lambda i: (0, 0))]
    )
    fs = pl.pallas_call(
        functools.partial(_mlp_body, nn, nblk),
        grid=(nblk,),
        in_specs=in_specs,
        out_specs=pl.BlockSpec((row_block, 1), lambda i: (i, 0)),
        out_shape=jax.ShapeDtypeStruct((T, 1), jnp.float32),
    )(*([g_perm] * nn + [tbl, W1, w2[:, None]]))
    return fs[:, 0]


def kernel(user_number_triangles, graph_nodes, graph_adjacency_matrix,
           W_gnn, w_out, W_dev, W_mlp1, w_mlp2):
    n = graph_nodes.shape[0]
    src = graph_adjacency_matrix[0]
    dst = graph_adjacency_matrix[1]
    diff = graph_nodes[src] - graph_nodes[dst]
    h = jax.nn.relu(diff @ W_gnn)
    agg = jax.ops.segment_max(h, dst, num_segments=n)
    agg = jnp.where(jnp.isfinite(agg), agg, 0.0)
    inclusion_score = agg @ w_out
    target = min(n, U_TRI_K * 3)
    _, sel_idx = jax.lax.top_k(inclusion_score, target)
    ext = graph_nodes[sel_idx]

    neigh = _knn_pallas(jax.lax.stop_gradient(ext), K_KNN_K, 64)

    dif2 = ext[:, None, :] - ext[neigh]
    h2 = jax.nn.relu(dif2 @ W_dev)
    edge_feat = jnp.max(h2, axis=1)
    f = jnp.mean(edge_feat, axis=1)
    logits = f[:, None] + f[neigh]
    S = jax.nn.softmax(logits, axis=1)
    A_s = S / (jnp.sum(S, axis=1, keepdims=True) + 1e-12)
    idx_i = jnp.repeat(jnp.arange(target), T_PER_K)
    m_t = jnp.tile(jnp.arange(T_PER_K), target)
    tri_j = neigh[idx_i, m_t]
    tri_k = neigh[idx_i, m_t + 1]
    tri_ids = jnp.stack([idx_i, tri_j, tri_k], axis=1)
    triangles = ext[tri_ids]
    p_init = A_s[idx_i, m_t] * A_s[idx_i, m_t + 1]
    bary = jnp.mean(triangles, axis=1)

    n_idx = _knn_pallas(jax.lax.stop_gradient(bary), N_NEIGH_TRI_K, 64)

    T = tri_ids.shape[0]
    tbl = jnp.concatenate(
        [bary, triangles.reshape(T, 9), p_init[:, None],
         jnp.zeros((T, 3), jnp.float32)], axis=1)
    g_perm = tbl[n_idx.T.reshape(-1)]
    final_scores = _mlp_scores_pallas(tbl, g_perm, W_mlp1, w_mlp2, N_NEIGH_TRI_K)
    _, top_tri = jax.lax.top_k(final_scores, U_TRI_K)
    top_tri = top_tri + 0 * user_number_triangles
    return triangles[top_tri]


# speedup vs baseline: 3.8087x; 1.0004x over previous
import functools

import jax
import jax.numpy as jnp
from jax import lax
from jax.experimental import pallas as pl
from jax.experimental.pallas import tpu as pltpu

T_PER_K = 4
N_NEIGH_TRI_K = 20
K_KNN_K = 15
U_TRI_K = 2000

_SENTINEL = 1.0e6


def _knn_body(k, n_real, points_ref, coords_t_ref, qsq_ref, rsq_ref, out_ref,
              d_ref):
    R = points_ref.shape[0]
    npad = coords_t_ref.shape[1]
    i = pl.program_id(0)

    p = points_ref[...]
    ct = coords_t_ref[...]
    dotp = lax.dot_general(p, ct, (((1,), (0,)), ((), ())),
                           preferred_element_type=jnp.float32)
    d = (qsq_ref[...] + rsq_ref[...]) - 2.0 * dotp

    cols = lax.broadcasted_iota(jnp.int32, (R, npad), 1)
    rows_g = i * R + lax.broadcasted_iota(jnp.int32, (R, npad), 0)
    d_ref[...] = jnp.where(cols == rows_g, jnp.inf, d)

    for t in range(k):
        dcur = d_ref[...]
        m = jnp.min(dcur, axis=1, keepdims=True)
        sel = jnp.where(dcur == m, cols, npad)
        idx = jnp.min(sel, axis=1, keepdims=True)
        out_ref[:, t:t + 1] = idx
        if t + 1 < k:
            d_ref[...] = jnp.where(cols == idx, jnp.inf, dcur)


def _knn_pallas(points, k, row_block):
    n = points.shape[0]
    npad = ((n + 127) // 128) * 128
    if npad % row_block != 0:
        npad = ((npad + row_block - 1) // row_block) * row_block
    pts = jnp.full((npad, 3), _SENTINEL, dtype=jnp.float32).at[:n].set(points)
    coords_t = pts.T
    sq = jnp.sum(pts * pts, axis=1)
    qsq = sq[:, None]
    rsq = sq[None, :]

    grid = npad // row_block
    out = pl.pallas_call(
        functools.partial(_knn_body, k, n),
        grid=(grid,),
        in_specs=[
            pl.BlockSpec((row_block, 3), lambda i: (i, 0)),
            pl.BlockSpec((3, npad), lambda i: (0, 0)),
            pl.BlockSpec((row_block, 1), lambda i: (i, 0)),
            pl.BlockSpec((1, npad), lambda i: (0, 0)),
        ],
        out_specs=pl.BlockSpec((row_block, k), lambda i: (i, 0)),
        out_shape=jax.ShapeDtypeStruct((npad, k), jnp.int32),
        scratch_shapes=[pltpu.VMEM((row_block, npad), jnp.float32)],
    )(pts, coords_t, qsq, rsq)
    return out[:n]


def _mlp_body(nn, nblk, *refs):
    g_refs = refs[:nn]
    tbl_ref, W1_ref, w2_ref, fs_ref = refs[nn], refs[nn + 1], refs[nn + 2], refs[nn + 3]
    self16 = tbl_ref[...]
    self12 = self16[:, 0:12]
    selfp = self16[:, 12:13]
    W1 = W1_ref[...]
    s = None
    for j in range(nn):
        g = g_refs[j][...]
        r_j = self12 - g[:, 0:12]
        hm = jnp.maximum(lax.dot_general(r_j, W1, (((1,), (0,)), ((), ())),
                                         preferred_element_type=jnp.float32), 0.0)
        wm = hm * g[:, 12:13]
        s = wm if s is None else s + wm
    aggm = s / float(nn)
    dv = lax.dot_general(aggm, w2_ref[...], (((1,), (0,)), ((), ())),
                         preferred_element_type=jnp.float32)
    fs_ref[...] = jax.nn.sigmoid(dv) * selfp


def _mlp_scores_pallas(tbl, g_perm, W1, w2, nn, row_block=1200):
    T = tbl.shape[0]
    nblk = T // row_block
    in_specs = (
        [pl.BlockSpec((row_block, 16), functools.partial(lambda j, i: (j * nblk + i, 0), j))
         for j in range(nn)]
        + [pl.BlockSpec((row_block, 16), lambda i: (i, 0)),
           pl.BlockSpec((12, 128), lambda i: (0, 0)),
           pl.BlockSpec((128, 1), lambda i: (0, 0))]
    )
    fs = pl.pallas_call(
        functools.partial(_mlp_body, nn, nblk),
        grid=(nblk,),
        in_specs=in_specs,
        out_specs=pl.BlockSpec((row_block, 1), lambda i: (i, 0)),
        out_shape=jax.ShapeDtypeStruct((T, 1), jnp.float32),
    )(*([g_perm] * nn + [tbl, W1, w2[:, None]]))
    return fs[:, 0]


def kernel(user_number_triangles, graph_nodes, graph_adjacency_matrix,
           W_gnn, w_out, W_dev, W_mlp1, w_mlp2):
    n = graph_nodes.shape[0]
    src = graph_adjacency_matrix[0]
    dst = graph_adjacency_matrix[1]
    diff = graph_nodes[src] - graph_nodes[dst]
    h = jax.nn.relu(diff @ W_gnn)
    agg = jax.ops.segment_max(h, dst, num_segments=n)
    agg = jnp.where(jnp.isfinite(agg), agg, 0.0)
    inclusion_score = agg @ w_out
    target = min(n, U_TRI_K * 3)
    _, sel_idx = jax.lax.top_k(inclusion_score, target)
    ext = graph_nodes[sel_idx]

    neigh = _knn_pallas(jax.lax.stop_gradient(ext), K_KNN_K, 64)

    dif2 = ext[:, None, :] - ext[neigh]
    h2 = jax.nn.relu(dif2 @ W_dev)
    edge_feat = jnp.max(h2, axis=1)
    f = jnp.mean(edge_feat, axis=1)
    logits = f[:, None] + f[neigh]
    S = jax.nn.softmax(logits, axis=1)
    A_s = S / (jnp.sum(S, axis=1, keepdims=True) + 1e-12)
    idx_i = jnp.repeat(jnp.arange(target), T_PER_K)
    m_t = jnp.tile(jnp.arange(T_PER_K), target)
    tri_j = neigh[idx_i, m_t]
    tri_k = neigh[idx_i, m_t + 1]
    tri_ids = jnp.stack([idx_i, tri_j, tri_k], axis=1)
    triangles = ext[tri_ids]
    p_init = A_s[idx_i, m_t] * A_s[idx_i, m_t + 1]
    bary = jnp.mean(triangles, axis=1)

    n_idx = _knn_pallas(jax.lax.stop_gradient(bary), N_NEIGH_TRI_K, 64)

    T = tri_ids.shape[0]
    tbl = jnp.concatenate(
        [bary, triangles.reshape(T, 9), p_init[:, None],
         jnp.zeros((T, 3), jnp.float32)], axis=1)
    g_perm = tbl[n_idx.T.reshape(-1)]
    final_scores = _mlp_scores_pallas(tbl, g_perm, W_mlp1, w_mlp2, N_NEIGH_TRI_K)
    _, top_tri = jax.lax.top_k(final_scores, U_TRI_K)
    top_tri = top_tri + 0 * user_number_triangles
    return triangles[top_tri]
